# Initial kernel scaffold; baseline (speedup 1.0000x reference)
import jax
import jax.numpy as jnp
from jax.experimental import pallas as pl


def kernel(x_0, incidence_1, W_init, b_init, W0, W1):
    raise NotImplementedError("write your pallas kernel here")


# speedup vs baseline: 1.2549x; 1.2549x over previous
import functools
import math

import jax
import jax.numpy as jnp
from jax import lax
from jax.experimental import pallas as pl

ALPHA = 0.5
F32 = jnp.float32


def _dn(cl, cr):
    return (((cl,), (cr,)), ((), ()))


def _p2_body(x0_ref, inc_ref, wi_ref, b_ref,
             x_ref, nd_ref, esz_ref, edn_ref, x1_ref):
    i = pl.program_id(0)
    x_blk = jnp.maximum(
        lax.dot_general(x0_ref[...], wi_ref[...], _dn(1, 1),
                        preferred_element_type=F32) + b_ref[...], 0.0)
    x_ref[...] = x_blk
    inc_blk = inc_ref[...]
    nd_blk = jnp.sum(inc_blk, axis=1, keepdims=True)
    nd_ref[...] = nd_blk

    @pl.when(i == 0)
    def _():
        esz_ref[...] = jnp.zeros_like(esz_ref)
        edn_ref[...] = jnp.zeros_like(edn_ref)
        x1_ref[...] = jnp.zeros_like(x1_ref)

    esz_ref[...] += jnp.sum(inc_blk, axis=0, keepdims=True)
    edn_ref[...] += lax.dot_general(nd_blk, inc_blk, _dn(0, 0),
                                    preferred_element_type=F32)
    x1_ref[...] += lax.dot_general(inc_blk, x_blk, _dn(0, 0),
                                   preferred_element_type=F32)


def _layer_body(inc_ref, z_ref, xskip_ref, rsnd_ref, w_ref, out_ref, *, beta):
    m = lax.dot_general(inc_ref[...], z_ref[...], _dn(1, 0),
                        preferred_element_type=F32)
    m = m * rsnd_ref[...]
    xc = (1.0 - ALPHA) * m + ALPHA * xskip_ref[...]
    out = (1.0 - beta) * xc + beta * lax.dot_general(
        xc, w_ref[...], _dn(1, 1), preferred_element_type=F32)
    out_ref[...] = jnp.maximum(out, 0.0)


def _p4_body(inc_ref, x2_ref, x1_ref):
    i = pl.program_id(0)

    @pl.when(i == 0)
    def _():
        x1_ref[...] = jnp.zeros_like(x1_ref)

    x1_ref[...] += lax.dot_general(inc_ref[...], x2_ref[...], _dn(0, 0),
                                   preferred_element_type=F32)


def _pick_bn(n):
    for bn in (200, 128, 80, 64, 40, 16, 8):
        if n % bn == 0:
            return bn
    return n


def kernel(x_0, incidence_1, W_init, b_init, W0, W1):
    N, D = x_0.shape
    E = incidence_1.shape[1]
    H = W_init.shape[0]
    BN = _pick_bn(N)
    nsteps = N // BN
    b2 = b_init.reshape(1, H)

    x, nd, esz_row, edn_row, x1_raw = pl.pallas_call(
        _p2_body,
        grid=(nsteps,),
        in_specs=[
            pl.BlockSpec((BN, D), lambda i: (i, 0)),
            pl.BlockSpec((BN, E), lambda i: (i, 0)),
            pl.BlockSpec((H, D), lambda i: (0, 0)),
            pl.BlockSpec((1, H), lambda i: (0, 0)),
        ],
        out_specs=[
            pl.BlockSpec((BN, H), lambda i: (i, 0)),
            pl.BlockSpec((BN, 1), lambda i: (i, 0)),
            pl.BlockSpec((1, E), lambda i: (0, 0)),
            pl.BlockSpec((1, E), lambda i: (0, 0)),
            pl.BlockSpec((E, H), lambda i: (0, 0)),
        ],
        out_shape=[
            jax.ShapeDtypeStruct((N, H), F32),
            jax.ShapeDtypeStruct((N, 1), F32),
            jax.ShapeDtypeStruct((1, E), F32),
            jax.ShapeDtypeStruct((1, E), F32),
            jax.ShapeDtypeStruct((E, H), F32),
        ],
    )(x_0, incidence_1, W_init, b2)

    esz = esz_row.reshape(E, 1)
    rsqe = lax.rsqrt(edn_row.reshape(E, 1) / esz)
    z1 = x1_raw * (rsqe / esz)
    rsnd = lax.rsqrt(nd)

    def layer(z, x_skip, W, beta):
        return pl.pallas_call(
            functools.partial(_layer_body, beta=beta),
            grid=(nsteps,),
            in_specs=[
                pl.BlockSpec((BN, E), lambda i: (i, 0)),
                pl.BlockSpec((E, H), lambda i: (0, 0)),
                pl.BlockSpec((BN, H), lambda i: (i, 0)),
                pl.BlockSpec((BN, 1), lambda i: (i, 0)),
                pl.BlockSpec((H, H), lambda i: (0, 0)),
            ],
            out_specs=pl.BlockSpec((BN, H), lambda i: (i, 0)),
            out_shape=jax.ShapeDtypeStruct((N, H), F32),
        )(incidence_1, z, x_skip, rsnd, W)

    beta1 = math.log(ALPHA / 1.0 + 1.0)
    beta2 = math.log(ALPHA / 2.0 + 1.0)

    x2 = layer(z1, x, W0, beta1)

    x1_raw2 = pl.pallas_call(
        _p4_body,
        grid=(nsteps,),
        in_specs=[
            pl.BlockSpec((BN, E), lambda i: (i, 0)),
            pl.BlockSpec((BN, H), lambda i: (i, 0)),
        ],
        out_specs=pl.BlockSpec((E, H), lambda i: (0, 0)),
        out_shape=jax.ShapeDtypeStruct((E, H), F32),
    )(incidence_1, x2)

    x1_out = x1_raw2 / esz
    z2 = x1_out * rsqe
    x_out = layer(z2, x, W1, beta2)
    return x_out, x1_out


# speedup vs baseline: 1.2751x; 1.0161x over previous
import functools
import math

import jax
import jax.numpy as jnp
from jax import lax
from jax.experimental import pallas as pl

ALPHA = 0.5
F32 = jnp.float32


def _dn(cl, cr):
    return (((cl,), (cr,)), ((), ()))


def _p2_body(x0_ref, inc_ref, wi_ref, b_ref,
             x_ref, nd_ref, esz_ref, edn_ref, x1_ref, pk_ref, *, E, Q):
    i = pl.program_id(0)
    x_blk = jnp.maximum(
        lax.dot_general(x0_ref[...], wi_ref[...], _dn(1, 1),
                        preferred_element_type=F32) + b_ref[...], 0.0)
    x_ref[...] = x_blk
    inc_blk = inc_ref[...]
    nd_blk = jnp.sum(inc_blk, axis=1, keepdims=True)
    nd_ref[...] = nd_blk

    @pl.when(i == 0)
    def _():
        esz_ref[...] = jnp.zeros_like(esz_ref)
        edn_ref[...] = jnp.zeros_like(edn_ref)
        x1_ref[...] = jnp.zeros_like(x1_ref)

    esz_ref[...] += jnp.sum(inc_blk, axis=0, keepdims=True)
    edn_ref[...] += lax.dot_general(nd_blk, inc_blk, _dn(0, 0),
                                    preferred_element_type=F32)

    pad = 4 * Q - E
    if pad:
        incp = jnp.concatenate(
            [inc_blk, jnp.zeros((inc_blk.shape[0], pad), F32)], axis=1)
    else:
        incp = inc_blk
    pk = jnp.zeros(incp.shape[:1] + (Q,), jnp.int32)
    for k in range(4):
        qk = incp[:, k * Q:(k + 1) * Q]
        pk = pk | (qk.astype(jnp.int32) << (8 * k))
        x1_ref[k * Q:(k + 1) * Q, :] += lax.dot_general(
            qk, x_blk, _dn(0, 0), preferred_element_type=F32)
    pk_ref[...] = pk


def _p34_body(pk_ref, z1_ref, xskip_ref, rsnd_ref, w_ref,
              x2_ref, x1b_ref, *, beta, Q):
    i = pl.program_id(0)
    pk = pk_ref[...]
    qs = []
    m = jnp.zeros(x2_ref.shape, F32)
    for k in range(4):
        qk = ((pk >> (8 * k)) & 0xFF).astype(F32)
        qs.append(qk)
        m += lax.dot_general(qk, z1_ref[k * Q:(k + 1) * Q, :], _dn(1, 0),
                             preferred_element_type=F32)
    m = m * rsnd_ref[...]
    xc = (1.0 - ALPHA) * m + ALPHA * xskip_ref[...]
    out = (1.0 - beta) * xc + beta * lax.dot_general(
        xc, w_ref[...], _dn(1, 1), preferred_element_type=F32)
    x2 = jnp.maximum(out, 0.0)
    x2_ref[...] = x2

    @pl.when(i == 0)
    def _():
        x1b_ref[...] = jnp.zeros_like(x1b_ref)

    for k in range(4):
        x1b_ref[k * Q:(k + 1) * Q, :] += lax.dot_general(
            qs[k], x2, _dn(0, 0), preferred_element_type=F32)


def _p5_body(pk_ref, z_ref, xskip_ref, rsnd_ref, w_ref, out_ref, *, beta, Q):
    pk = pk_ref[...]
    m = jnp.zeros(out_ref.shape, F32)
    for k in range(4):
        qk = ((pk >> (8 * k)) & 0xFF).astype(F32)
        m += lax.dot_general(qk, z_ref[k * Q:(k + 1) * Q, :], _dn(1, 0),
                             preferred_element_type=F32)
    m = m * rsnd_ref[...]
    xc = (1.0 - ALPHA) * m + ALPHA * xskip_ref[...]
    out = (1.0 - beta) * xc + beta * lax.dot_general(
        xc, w_ref[...], _dn(1, 1), preferred_element_type=F32)
    out_ref[...] = jnp.maximum(out, 0.0)


def _pick_bn(n):
    for bn in (200, 128, 80, 64, 40, 16, 8):
        if n % bn == 0:
            return bn
    return n


def kernel(x_0, incidence_1, W_init, b_init, W0, W1):
    N, D = x_0.shape
    E = incidence_1.shape[1]
    H = W_init.shape[0]
    BN = _pick_bn(N)
    nsteps = N // BN
    Q = -(-E // (4 * 128)) * 128
    EP = 4 * Q
    b2 = b_init.reshape(1, H)

    x, nd, esz_row, edn_row, x1p, packed = pl.pallas_call(
        functools.partial(_p2_body, E=E, Q=Q),
        grid=(nsteps,),
        in_specs=[
            pl.BlockSpec((BN, D), lambda i: (i, 0)),
            pl.BlockSpec((BN, E), lambda i: (i, 0)),
            pl.BlockSpec((H, D), lambda i: (0, 0)),
            pl.BlockSpec((1, H), lambda i: (0, 0)),
        ],
        out_specs=[
            pl.BlockSpec((BN, H), lambda i: (i, 0)),
            pl.BlockSpec((BN, 1), lambda i: (i, 0)),
            pl.BlockSpec((1, E), lambda i: (0, 0)),
            pl.BlockSpec((1, E), lambda i: (0, 0)),
            pl.BlockSpec((EP, H), lambda i: (0, 0)),
            pl.BlockSpec((BN, Q), lambda i: (i, 0)),
        ],
        out_shape=[
            jax.ShapeDtypeStruct((N, H), F32),
            jax.ShapeDtypeStruct((N, 1), F32),
            jax.ShapeDtypeStruct((1, E), F32),
            jax.ShapeDtypeStruct((1, E), F32),
            jax.ShapeDtypeStruct((EP, H), F32),
            jax.ShapeDtypeStruct((N, Q), jnp.int32),
        ],
    )(x_0, incidence_1, W_init, b2)

    esz = esz_row.reshape(E, 1)
    rsqe = lax.rsqrt(edn_row.reshape(E, 1) / esz)
    scale1 = jnp.zeros((EP, 1), F32).at[:E].set(rsqe / esz)
    z1 = x1p * scale1
    rsnd = lax.rsqrt(nd)

    beta1 = math.log(ALPHA / 1.0 + 1.0)
    beta2 = math.log(ALPHA / 2.0 + 1.0)

    x2, x1p2 = pl.pallas_call(
        functools.partial(_p34_body, beta=beta1, Q=Q),
        grid=(nsteps,),
        in_specs=[
            pl.BlockSpec((BN, Q), lambda i: (i, 0)),
            pl.BlockSpec((EP, H), lambda i: (0, 0)),
            pl.BlockSpec((BN, H), lambda i: (i, 0)),
            pl.BlockSpec((BN, 1), lambda i: (i, 0)),
            pl.BlockSpec((H, H), lambda i: (0, 0)),
        ],
        out_specs=[
            pl.BlockSpec((BN, H), lambda i: (i, 0)),
            pl.BlockSpec((EP, H), lambda i: (0, 0)),
        ],
        out_shape=[
            jax.ShapeDtypeStruct((N, H), F32),
            jax.ShapeDtypeStruct((EP, H), F32),
        ],
    )(packed, z1, x, rsnd, W0)

    x1_out = x1p2[:E] / esz
    z2 = x1p2 * jnp.zeros((EP, 1), F32).at[:E].set(rsqe / esz)

    x_out = pl.pallas_call(
        functools.partial(_p5_body, beta=beta2, Q=Q),
        grid=(nsteps,),
        in_specs=[
            pl.BlockSpec((BN, Q), lambda i: (i, 0)),
            pl.BlockSpec((EP, H), lambda i: (0, 0)),
            pl.BlockSpec((BN, H), lambda i: (i, 0)),
            pl.BlockSpec((BN, 1), lambda i: (i, 0)),
            pl.BlockSpec((H, H), lambda i: (0, 0)),
        ],
        out_specs=pl.BlockSpec((BN, H), lambda i: (i, 0)),
        out_shape=jax.ShapeDtypeStruct((N, H), F32),
    )(packed, z2, x, rsnd, W1)

    return x_out, x1_out


# speedup vs baseline: 1.5094x; 1.1837x over previous
import functools
import math

import jax
import jax.numpy as jnp
from jax import lax
from jax.experimental import pallas as pl

ALPHA = 0.5
F32 = jnp.float32
BF16 = jnp.bfloat16


def _dn(cl, cr):
    return (((cl,), (cr,)), ((), ()))


def _p2_body(x0_ref, inc_ref, wi_ref, b_ref,
             x_ref, nd_ref, esz_ref, edn_ref, x1_ref, pk_ref, *, E, Q):
    i = pl.program_id(0)
    x_blk = jnp.maximum(
        lax.dot_general(x0_ref[...], wi_ref[...], _dn(1, 1),
                        preferred_element_type=F32) + b_ref[...], 0.0)
    x_ref[...] = x_blk
    inc_blk = inc_ref[...]
    nd_blk = jnp.sum(inc_blk, axis=1, keepdims=True)
    nd_ref[...] = nd_blk

    @pl.when(i == 0)
    def _():
        esz_ref[...] = jnp.zeros_like(esz_ref)
        edn_ref[...] = jnp.zeros_like(edn_ref)
        x1_ref[...] = jnp.zeros_like(x1_ref)

    esz_ref[...] += jnp.sum(inc_blk, axis=0, keepdims=True)
    edn_ref[...] += lax.dot_general(nd_blk, inc_blk, _dn(0, 0),
                                    preferred_element_type=F32)

    pad = 4 * Q - E
    if pad:
        incp = jnp.concatenate(
            [inc_blk, jnp.zeros((inc_blk.shape[0], pad), F32)], axis=1)
    else:
        incp = inc_blk
    pk = jnp.zeros(incp.shape[:1] + (Q,), jnp.int32)
    xb = x_blk.astype(BF16)
    for k in range(4):
        qk = incp[:, k * Q:(k + 1) * Q]
        pk = pk | (qk.astype(jnp.int32) << (8 * k))
        x1_ref[k * Q:(k + 1) * Q, :] += lax.dot_general(
            qk.astype(BF16), xb, _dn(0, 0), preferred_element_type=F32)
    pk_ref[...] = pk


def _p34_body(pk_ref, z1_ref, xskip_ref, rsnd_ref, w_ref,
              x2_ref, x1b_ref, *, beta, Q):
    i = pl.program_id(0)
    pk = pk_ref[...]
    qs = []
    m = jnp.zeros(x2_ref.shape, F32)
    for k in range(4):
        qk = ((pk >> (8 * k)) & 0xFF).astype(F32).astype(BF16)
        qs.append(qk)
        m += lax.dot_general(qk, z1_ref[k * Q:(k + 1) * Q, :], _dn(1, 0),
                             preferred_element_type=F32)
    m = m * rsnd_ref[...]
    xc = (1.0 - ALPHA) * m + ALPHA * xskip_ref[...]
    out = (1.0 - beta) * xc + beta * lax.dot_general(
        xc, w_ref[...], _dn(1, 1), preferred_element_type=F32)
    x2 = jnp.maximum(out, 0.0)
    x2_ref[...] = x2

    @pl.when(i == 0)
    def _():
        x1b_ref[...] = jnp.zeros_like(x1b_ref)

    x2b = x2.astype(BF16)
    for k in range(4):
        x1b_ref[k * Q:(k + 1) * Q, :] += lax.dot_general(
            qs[k], x2b, _dn(0, 0), preferred_element_type=F32)


def _p5_body(pk_ref, z_ref, xskip_ref, rsnd_ref, w_ref, out_ref, *, beta, Q):
    pk = pk_ref[...]
    m = jnp.zeros(out_ref.shape, F32)
    for k in range(4):
        qk = ((pk >> (8 * k)) & 0xFF).astype(F32).astype(BF16)
        m += lax.dot_general(qk, z_ref[k * Q:(k + 1) * Q, :], _dn(1, 0),
                             preferred_element_type=F32)
    m = m * rsnd_ref[...]
    xc = (1.0 - ALPHA) * m + ALPHA * xskip_ref[...]
    out = (1.0 - beta) * xc + beta * lax.dot_general(
        xc, w_ref[...], _dn(1, 1), preferred_element_type=F32)
    out_ref[...] = jnp.maximum(out, 0.0)


def _pick_bn(n):
    for bn in (200, 128, 80, 64, 40, 16, 8):
        if n % bn == 0:
            return bn
    return n


def kernel(x_0, incidence_1, W_init, b_init, W0, W1):
    N, D = x_0.shape
    E = incidence_1.shape[1]
    H = W_init.shape[0]
    BN = _pick_bn(N)
    nsteps = N // BN
    Q = -(-E // (4 * 128)) * 128
    EP = 4 * Q
    b2 = b_init.reshape(1, H)

    x, nd, esz_row, edn_row, x1p, packed = pl.pallas_call(
        functools.partial(_p2_body, E=E, Q=Q),
        grid=(nsteps,),
        in_specs=[
            pl.BlockSpec((BN, D), lambda i: (i, 0)),
            pl.BlockSpec((BN, E), lambda i: (i, 0)),
            pl.BlockSpec((H, D), lambda i: (0, 0)),
            pl.BlockSpec((1, H), lambda i: (0, 0)),
        ],
        out_specs=[
            pl.BlockSpec((BN, H), lambda i: (i, 0)),
            pl.BlockSpec((BN, 1), lambda i: (i, 0)),
            pl.BlockSpec((1, E), lambda i: (0, 0)),
            pl.BlockSpec((1, E), lambda i: (0, 0)),
            pl.BlockSpec((EP, H), lambda i: (0, 0)),
            pl.BlockSpec((BN, Q), lambda i: (i, 0)),
        ],
        out_shape=[
            jax.ShapeDtypeStruct((N, H), F32),
            jax.ShapeDtypeStruct((N, 1), F32),
            jax.ShapeDtypeStruct((1, E), F32),
            jax.ShapeDtypeStruct((1, E), F32),
            jax.ShapeDtypeStruct((EP, H), F32),
            jax.ShapeDtypeStruct((N, Q), jnp.int32),
        ],
    )(x_0, incidence_1, W_init, b2)

    esz = esz_row.reshape(E, 1)
    rsqe = lax.rsqrt(edn_row.reshape(E, 1) / esz)
    scale1 = jnp.zeros((EP, 1), F32).at[:E].set(rsqe / esz)
    z1 = (x1p * scale1).astype(BF16)
    rsnd = lax.rsqrt(nd)

    beta1 = math.log(ALPHA / 1.0 + 1.0)
    beta2 = math.log(ALPHA / 2.0 + 1.0)

    x2, x1p2 = pl.pallas_call(
        functools.partial(_p34_body, beta=beta1, Q=Q),
        grid=(nsteps,),
        in_specs=[
            pl.BlockSpec((BN, Q), lambda i: (i, 0)),
            pl.BlockSpec((EP, H), lambda i: (0, 0)),
            pl.BlockSpec((BN, H), lambda i: (i, 0)),
            pl.BlockSpec((BN, 1), lambda i: (i, 0)),
            pl.BlockSpec((H, H), lambda i: (0, 0)),
        ],
        out_specs=[
            pl.BlockSpec((BN, H), lambda i: (i, 0)),
            pl.BlockSpec((EP, H), lambda i: (0, 0)),
        ],
        out_shape=[
            jax.ShapeDtypeStruct((N, H), F32),
            jax.ShapeDtypeStruct((EP, H), F32),
        ],
    )(packed, z1, x, rsnd, W0)

    x1_out = x1p2[:E] / esz
    z2 = (x1p2 * jnp.zeros((EP, 1), F32).at[:E].set(rsqe / esz)).astype(BF16)

    x_out = pl.pallas_call(
        functools.partial(_p5_body, beta=beta2, Q=Q),
        grid=(nsteps,),
        in_specs=[
            pl.BlockSpec((BN, Q), lambda i: (i, 0)),
            pl.BlockSpec((EP, H), lambda i: (0, 0)),
            pl.BlockSpec((BN, H), lambda i: (i, 0)),
            pl.BlockSpec((BN, 1), lambda i: (i, 0)),
            pl.BlockSpec((H, H), lambda i: (0, 0)),
        ],
        out_specs=pl.BlockSpec((BN, H), lambda i: (i, 0)),
        out_shape=jax.ShapeDtypeStruct((N, H), F32),
    )(packed, z2, x, rsnd, W1)

    return x_out, x1_out


# speedup vs baseline: 1.6210x; 1.0740x over previous
import functools
import math

import jax
import jax.numpy as jnp
from jax import lax
from jax.experimental import pallas as pl

ALPHA = 0.5
F32 = jnp.float32
BF16 = jnp.bfloat16


def _dn(cl, cr):
    return (((cl,), (cr,)), ((), ()))


def _p2_body(x0_ref, inc_ref, wi_ref, b_ref,
             x_ref, nd_ref, deg_ref, x1t_ref, pk_ref, *, E, Q):
    i = pl.program_id(0)
    x_blk = jnp.maximum(
        lax.dot_general(x0_ref[...], wi_ref[...], _dn(1, 1),
                        preferred_element_type=F32) + b_ref[...], 0.0)
    x_ref[...] = x_blk
    inc_blk = inc_ref[...]
    nd_blk = jnp.sum(inc_blk, axis=1, keepdims=True)
    nd_ref[...] = nd_blk

    @pl.when(i == 0)
    def _():
        deg_ref[...] = jnp.zeros_like(deg_ref)
        x1t_ref[...] = jnp.zeros_like(x1t_ref)

    lhs = jnp.concatenate([jnp.ones_like(nd_blk), nd_blk], axis=1)
    deg_ref[...] += lax.dot_general(lhs, inc_blk, _dn(0, 0),
                                    preferred_element_type=F32)

    pad = 4 * Q - E
    if pad:
        incp = jnp.concatenate(
            [inc_blk, jnp.zeros((inc_blk.shape[0], pad), F32)], axis=1)
    else:
        incp = inc_blk
    pk = jnp.zeros(incp.shape[:1] + (Q,), jnp.int32)
    xb = x_blk.astype(BF16)
    for k in range(4):
        qk = incp[:, k * Q:(k + 1) * Q]
        pk = pk | (qk.astype(jnp.int32) << (8 * k))
        x1t_ref[:, k * Q:(k + 1) * Q] += lax.dot_general(
            xb, qk.astype(BF16), _dn(0, 0), preferred_element_type=F32)
    pk_ref[...] = pk


def _p34_body(pk_ref, z1_ref, xskip_ref, rsnd_ref, w_ref,
              x2_ref, x1t_ref, *, beta, Q):
    i = pl.program_id(0)
    pk = pk_ref[...]
    qs = []
    m = jnp.zeros(x2_ref.shape, F32)
    for k in range(4):
        qk = ((pk >> (8 * k)) & 0xFF).astype(F32).astype(BF16)
        qs.append(qk)
        m += lax.dot_general(qk, z1_ref[k * Q:(k + 1) * Q, :], _dn(1, 0),
                             preferred_element_type=F32)
    m = m * rsnd_ref[...]
    xc = (1.0 - ALPHA) * m + ALPHA * xskip_ref[...]
    out = (1.0 - beta) * xc + beta * lax.dot_general(
        xc, w_ref[...], _dn(1, 1), preferred_element_type=F32)
    x2 = jnp.maximum(out, 0.0)
    x2_ref[...] = x2

    @pl.when(i == 0)
    def _():
        x1t_ref[...] = jnp.zeros_like(x1t_ref)

    x2b = x2.astype(BF16)
    for k in range(4):
        x1t_ref[:, k * Q:(k + 1) * Q] += lax.dot_general(
            x2b, qs[k], _dn(0, 0), preferred_element_type=F32)


def _p5_body(pk_ref, z_ref, xskip_ref, rsnd_ref, w_ref, out_ref, *, beta, Q):
    pk = pk_ref[...]
    m = jnp.zeros(out_ref.shape, F32)
    for k in range(4):
        qk = ((pk >> (8 * k)) & 0xFF).astype(F32).astype(BF16)
        m += lax.dot_general(qk, z_ref[k * Q:(k + 1) * Q, :], _dn(1, 0),
                             preferred_element_type=F32)
    m = m * rsnd_ref[...]
    xc = (1.0 - ALPHA) * m + ALPHA * xskip_ref[...]
    out = (1.0 - beta) * xc + beta * lax.dot_general(
        xc, w_ref[...], _dn(1, 1), preferred_element_type=F32)
    out_ref[...] = jnp.maximum(out, 0.0)


def _pick_bn(n, cap):
    for bn in (400, 200, 128, 80, 64, 40, 16, 8):
        if bn <= cap and n % bn == 0:
            return bn
    return n


def kernel(x_0, incidence_1, W_init, b_init, W0, W1):
    N, D = x_0.shape
    E = incidence_1.shape[1]
    H = W_init.shape[0]
    BN2 = _pick_bn(N, 200)
    BN = _pick_bn(N, 400)
    Q = -(-E // (4 * 128)) * 128
    EP = 4 * Q
    b2 = b_init.reshape(1, H)

    x, nd, deg, x1t, packed = pl.pallas_call(
        functools.partial(_p2_body, E=E, Q=Q),
        grid=(N // BN2,),
        in_specs=[
            pl.BlockSpec((BN2, D), lambda i: (i, 0)),
            pl.BlockSpec((BN2, E), lambda i: (i, 0)),
            pl.BlockSpec((H, D), lambda i: (0, 0)),
            pl.BlockSpec((1, H), lambda i: (0, 0)),
        ],
        out_specs=[
            pl.BlockSpec((BN2, H), lambda i: (i, 0)),
            pl.BlockSpec((BN2, 1), lambda i: (i, 0)),
            pl.BlockSpec((2, E), lambda i: (0, 0)),
            pl.BlockSpec((H, EP), lambda i: (0, 0)),
            pl.BlockSpec((BN2, Q), lambda i: (i, 0)),
        ],
        out_shape=[
            jax.ShapeDtypeStruct((N, H), F32),
            jax.ShapeDtypeStruct((N, 1), F32),
            jax.ShapeDtypeStruct((2, E), F32),
            jax.ShapeDtypeStruct((H, EP), F32),
            jax.ShapeDtypeStruct((N, Q), jnp.int32),
        ],
    )(x_0, incidence_1, W_init, b2)

    esz_row = deg[0:1]
    rsqe_row = lax.rsqrt(deg[1:2] / esz_row)
    scale_row = jnp.zeros((1, EP), F32).at[:, :E].set(rsqe_row / esz_row)
    z1 = (x1t * scale_row).T.astype(BF16)
    rsnd = lax.rsqrt(nd)

    beta1 = math.log(ALPHA / 1.0 + 1.0)
    beta2 = math.log(ALPHA / 2.0 + 1.0)

    x2, x1t2 = pl.pallas_call(
        functools.partial(_p34_body, beta=beta1, Q=Q),
        grid=(N // BN,),
        in_specs=[
            pl.BlockSpec((BN, Q), lambda i: (i, 0)),
            pl.BlockSpec((EP, H), lambda i: (0, 0)),
            pl.BlockSpec((BN, H), lambda i: (i, 0)),
            pl.BlockSpec((BN, 1), lambda i: (i, 0)),
            pl.BlockSpec((H, H), lambda i: (0, 0)),
        ],
        out_specs=[
            pl.BlockSpec((BN, H), lambda i: (i, 0)),
            pl.BlockSpec((H, EP), lambda i: (0, 0)),
        ],
        out_shape=[
            jax.ShapeDtypeStruct((N, H), F32),
            jax.ShapeDtypeStruct((H, EP), F32),
        ],
    )(packed, z1, x, rsnd, W0)

    x1_out = (x1t2[:, :E] / esz_row).T
    z2 = (x1t2 * scale_row).T.astype(BF16)

    x_out = pl.pallas_call(
        functools.partial(_p5_body, beta=beta2, Q=Q),
        grid=(N // BN,),
        in_specs=[
            pl.BlockSpec((BN, Q), lambda i: (i, 0)),
            pl.BlockSpec((EP, H), lambda i: (0, 0)),
            pl.BlockSpec((BN, H), lambda i: (i, 0)),
            pl.BlockSpec((BN, 1), lambda i: (i, 0)),
            pl.BlockSpec((H, H), lambda i: (0, 0)),
        ],
        out_specs=pl.BlockSpec((BN, H), lambda i: (i, 0)),
        out_shape=jax.ShapeDtypeStruct((N, H), F32),
    )(packed, z2, x, rsnd, W1)

    return x_out, x1_out
